# baseline (speedup 1.0000x reference)
import functools
import jax
import jax.numpy as jnp
from jax import lax
from jax.experimental import pallas as pl
from jax.experimental.pallas import tpu as pltpu
from jax.experimental.pallas import tpu_sc as plsc

_CH = 16
_NSLOTS = 4


def _make_sc(batch, seq_len, d_model, dtype):
    info = plsc.get_sparse_core_info()
    nc, ns = info.num_cores, info.num_subcores
    nw = nc * ns
    rows_per_w = seq_len // nw
    nchunks = rows_per_w // _CH
    mesh = plsc.VectorSubcoreMesh(core_axis_name="c", subcore_axis_name="s")

    @functools.partial(
        pl.kernel,
        mesh=mesh,
        out_type=jax.ShapeDtypeStruct((batch * seq_len, d_model), dtype),
        scratch_types=[
            pltpu.VMEM((_NSLOTS, _CH, d_model), dtype),
            pltpu.SemaphoreType.DMA((_NSLOTS,)),
            pltpu.SemaphoreType.DMA((_NSLOTS,)),
        ],
    )
    def k(w_hbm, out_hbm, buf, insem, outsem):
        wid = lax.axis_index("s") * nc + lax.axis_index("c")
        base = wid * rows_per_w

        def in_copy(j, slot):
            r = base + j * _CH
            return pltpu.make_async_copy(
                w_hbm.at[pl.ds(r, _CH), :], buf.at[slot], insem.at[slot]
            )

        def out_copy(j, slot, b):
            r = base + j * _CH
            return pltpu.make_async_copy(
                buf.at[slot],
                out_hbm.at[pl.ds(b * seq_len + r, _CH), :],
                outsem.at[slot],
            )

        for j in range(min(_NSLOTS, nchunks)):
            in_copy(j, j).start()
        for j in range(nchunks):
            slot = j % _NSLOTS
            in_copy(j, slot).wait()
            for b in range(batch):
                out_copy(j, slot, b).start()
            nxt = j + _NSLOTS
            if nxt < nchunks:
                for b in range(batch):
                    out_copy(j, slot, b).wait()
                in_copy(nxt, slot).start()
        for j in range(max(0, nchunks - _NSLOTS), nchunks):
            for b in range(batch):
                out_copy(j, j % _NSLOTS, b).wait()

    return k


def kernel(tokens, W_pos):
    batch, seq_len = tokens.shape
    d_model = W_pos.shape[1]
    flat = _make_sc(batch, seq_len, d_model, W_pos.dtype)(W_pos)
    return flat.reshape(batch, seq_len, d_model)


# speedup vs baseline: 1.0428x; 1.0428x over previous
import functools
import jax
import jax.numpy as jnp
from jax import lax
from jax.experimental import pallas as pl
from jax.experimental.pallas import tpu as pltpu
from jax.experimental.pallas import tpu_sc as plsc

_CH = 32
_NSLOTS = 2


def _make_sc(batch, seq_len, d_model, dtype):
    info = plsc.get_sparse_core_info()
    nc, ns = info.num_cores, info.num_subcores
    nw = nc * ns
    rows_per_w = seq_len // nw
    nchunks = rows_per_w // _CH
    mesh = plsc.VectorSubcoreMesh(core_axis_name="c", subcore_axis_name="s")

    @functools.partial(
        pl.kernel,
        mesh=mesh,
        out_type=jax.ShapeDtypeStruct((batch * seq_len, d_model), dtype),
        scratch_types=[
            pltpu.VMEM((_NSLOTS, _CH, d_model), dtype),
            pltpu.SemaphoreType.DMA((_NSLOTS,)),
            pltpu.SemaphoreType.DMA((_NSLOTS,)),
        ],
    )
    def k(w_hbm, out_hbm, buf, insem, outsem):
        wid = lax.axis_index("s") * nc + lax.axis_index("c")
        base = wid * rows_per_w

        def in_copy(j, slot):
            r = base + j * _CH
            return pltpu.make_async_copy(
                w_hbm.at[pl.ds(r, _CH), :], buf.at[slot], insem.at[slot]
            )

        def out_copy(j, slot, b):
            r = base + j * _CH
            return pltpu.make_async_copy(
                buf.at[slot],
                out_hbm.at[pl.ds(b * seq_len + r, _CH), :],
                outsem.at[slot],
            )

        for j in range(min(_NSLOTS, nchunks)):
            in_copy(j, j).start()
        for j in range(nchunks):
            slot = j % _NSLOTS
            in_copy(j, slot).wait()
            for b in range(batch):
                out_copy(j, slot, b).start()
            nxt = j + _NSLOTS
            if nxt < nchunks:
                for b in range(batch):
                    out_copy(j, slot, b).wait()
                in_copy(nxt, slot).start()
        for j in range(max(0, nchunks - _NSLOTS), nchunks):
            for b in range(batch):
                out_copy(j, j % _NSLOTS, b).wait()

    return k


def kernel(tokens, W_pos):
    batch, seq_len = tokens.shape
    d_model = W_pos.shape[1]
    flat = _make_sc(batch, seq_len, d_model, W_pos.dtype)(W_pos)
    return flat.reshape(batch, seq_len, d_model)
